# speedup vs baseline: 15.2520x; 15.2520x over previous
import functools

import jax
import jax.numpy as jnp
from jax import lax
from jax.experimental import pallas as pl
from jax.experimental.pallas import tpu as pltpu
from jax.experimental.pallas import tpu_sc as plsc

N = 10000
E = 320000
D = 128
NP = 10240
ROWS_PER_TILE = NP // 16
C = 128
NCHUNKS = E // C
NW = 32
BASE_CH = NCHUNKS // NW
EXTRA = NCHUNKS - BASE_CH * NW


def _worker_chunks(c, s):
    wid = s * 2 + c
    nch = BASE_CH + jnp.where(wid < EXTRA, 1, 0)
    c0 = BASE_CH * wid + jnp.minimum(wid, EXTRA)
    return c0, nch


def _deg_body(ei_hbm, ones_hbm, zd_hbm, deg_out, didx, ones_v, deg_sh, sem):
    c = lax.axis_index("c")
    s = lax.axis_index("s")
    pltpu.sync_copy(ones_hbm, ones_v)
    pltpu.sync_copy(zd_hbm, deg_sh.at[pl.ds(s * ROWS_PER_TILE, ROWS_PER_TILE)])
    plsc.subcore_barrier()
    c0, nch = _worker_chunks(c, s)

    def body(g, carry):
        base = (c0 + g) * C
        pltpu.sync_copy(ei_hbm.at[1, pl.ds(base, C)], didx.at[0])
        pltpu.sync_copy(ones_v, deg_sh.at[didx.at[0]], add=True)
        return carry

    lax.fori_loop(0, nch, body, 0)
    plsc.subcore_barrier()
    sl = pl.ds(s * ROWS_PER_TILE, ROWS_PER_TILE)
    pltpu.sync_copy(deg_sh.at[sl], deg_out.at[c, sl])


def _agg_body(h_hbm, ei_hbm, z_hbm, acc_out, sidx, didx, rows, acc_sh, sem):
    c = lax.axis_index("c")
    s = lax.axis_index("s")
    for k in range(ROWS_PER_TILE // C):
        pltpu.sync_copy(z_hbm, acc_sh.at[pl.ds(s * ROWS_PER_TILE + k * C, C)])
    plsc.subcore_barrier()
    c0, nch = _worker_chunks(c, s)

    def body(g, carry):
        base = (c0 + g) * C
        pltpu.sync_copy(ei_hbm.at[0, pl.ds(base, C)], sidx)
        pltpu.sync_copy(ei_hbm.at[1, pl.ds(base, C)], didx.at[0])
        pltpu.async_copy(h_hbm.at[sidx], rows, sem).wait()
        pltpu.sync_copy(rows, acc_sh.at[didx.at[0]], add=True)
        return carry

    lax.fori_loop(0, nch, body, 0)
    plsc.subcore_barrier()
    sl = pl.ds(s * ROWS_PER_TILE, ROWS_PER_TILE)
    pltpu.sync_copy(acc_sh.at[sl], acc_out.at[c, sl])


def _sc_deg(ei, ones_hbm, zd_hbm):
    mesh = plsc.VectorSubcoreMesh(core_axis_name="c", subcore_axis_name="s")
    f = pl.kernel(
        _deg_body,
        out_type=jax.ShapeDtypeStruct((2, NP), jnp.float32),
        mesh=mesh,
        scratch_types=[
            pltpu.VMEM((1, C), jnp.int32),
            pltpu.VMEM((C,), jnp.float32),
            pltpu.VMEM_SHARED((NP,), jnp.float32),
            pltpu.SemaphoreType.DMA,
        ],
    )
    return f(ei, ones_hbm, zd_hbm)


def _sc_agg(h, ei, z_hbm):
    mesh = plsc.VectorSubcoreMesh(core_axis_name="c", subcore_axis_name="s")
    f = pl.kernel(
        _agg_body,
        out_type=jax.ShapeDtypeStruct((2, NP, D), jnp.float32),
        mesh=mesh,
        scratch_types=[
            pltpu.VMEM((C,), jnp.int32),
            pltpu.VMEM((1, C), jnp.int32),
            pltpu.VMEM((C, D), jnp.float32),
            pltpu.VMEM_SHARED((NP, D), jnp.float32),
            pltpu.SemaphoreType.DMA,
        ],
    )
    return f(h, ei, z_hbm)


BLK = 1024
GRID = NP // BLK


def _swish(v):
    return v * jax.nn.sigmoid(v)


def _k1_body(x_ref, w_ref, r_ref, o_ref):
    o_ref[...] = jnp.dot(x_ref[...], w_ref[...],
                         preferred_element_type=jnp.float32) * r_ref[...]


def _tc_h1(x, W1, r):
    return pl.pallas_call(
        _k1_body,
        grid=(GRID,),
        in_specs=[
            pl.BlockSpec((BLK, D), lambda i: (i, 0)),
            pl.BlockSpec((D, D), lambda i: (0, 0)),
            pl.BlockSpec((BLK, 1), lambda i: (i, 0)),
        ],
        out_specs=pl.BlockSpec((BLK, D), lambda i: (i, 0)),
        out_shape=jax.ShapeDtypeStruct((NP, D), jnp.float32),
    )(x, W1, r)


def _k3_body(acc_ref, h_ref, r_ref, b_ref, w_ref, o_ref):
    agg = (acc_ref[0] + acc_ref[1] + h_ref[...]) * r_ref[...]
    x2 = _swish(agg + b_ref[...])
    o_ref[...] = jnp.dot(x2, w_ref[...],
                         preferred_element_type=jnp.float32) * r_ref[...]


def _tc_mid(acc, h, r, b1, W2):
    return pl.pallas_call(
        _k3_body,
        grid=(GRID,),
        in_specs=[
            pl.BlockSpec((2, BLK, D), lambda i: (0, i, 0)),
            pl.BlockSpec((BLK, D), lambda i: (i, 0)),
            pl.BlockSpec((BLK, 1), lambda i: (i, 0)),
            pl.BlockSpec((1, D), lambda i: (0, 0)),
            pl.BlockSpec((D, D), lambda i: (0, 0)),
        ],
        out_specs=pl.BlockSpec((BLK, D), lambda i: (i, 0)),
        out_shape=jax.ShapeDtypeStruct((NP, D), jnp.float32),
    )(acc, h, r, b1, W2)


def _k5_body(acc_ref, h_ref, r_ref, b_ref, wd_ref, bd_ref, wo_ref, bo_ref,
             o_ref):
    agg = (acc_ref[0] + acc_ref[1] + h_ref[...]) * r_ref[...]
    x3 = _swish(agg + b_ref[...])
    g = _swish(jnp.dot(x3, wd_ref[...],
                       preferred_element_type=jnp.float32) + bd_ref[...])
    o_ref[...] = jax.nn.sigmoid(
        jnp.dot(g, wo_ref[...], preferred_element_type=jnp.float32)
        + bo_ref[...])


def _tc_head(acc, h, r, b2, Wd, bd, Wo, bo):
    return pl.pallas_call(
        _k5_body,
        grid=(GRID,),
        in_specs=[
            pl.BlockSpec((2, BLK, D), lambda i: (0, i, 0)),
            pl.BlockSpec((BLK, D), lambda i: (i, 0)),
            pl.BlockSpec((BLK, 1), lambda i: (i, 0)),
            pl.BlockSpec((1, D), lambda i: (0, 0)),
            pl.BlockSpec((D, 100), lambda i: (0, 0)),
            pl.BlockSpec((1, 100), lambda i: (0, 0)),
            pl.BlockSpec((100, 1), lambda i: (0, 0)),
            pl.BlockSpec((1, 1), lambda i: (0, 0)),
        ],
        out_specs=pl.BlockSpec((BLK, 1), lambda i: (i, 0)),
        out_shape=jax.ShapeDtypeStruct((NP, 1), jnp.float32),
    )(acc, h, r, b2, Wd, bd, Wo, bo)


def kernel(x, edge_index, W1, b1, W2, b2, Wd, bd, Wo, bo):
    x_pad = jnp.zeros((NP, D), jnp.float32).at[:N].set(x)
    ones_hbm = jnp.ones((C,), jnp.float32)
    zd_hbm = jnp.zeros((ROWS_PER_TILE,), jnp.float32)
    z_hbm = jnp.zeros((C, D), jnp.float32)

    deg2 = _sc_deg(edge_index, ones_hbm, zd_hbm)
    r = lax.rsqrt(deg2[0] + deg2[1] + 1.0)[:, None]

    h1 = _tc_h1(x_pad, W1, r)
    acc1 = _sc_agg(h1, edge_index, z_hbm)
    h2 = _tc_mid(acc1, h1, r, b1.reshape(1, D), W2)
    acc2 = _sc_agg(h2, edge_index, z_hbm)
    out = _tc_head(acc2, h2, r, b2.reshape(1, D), Wd, bd.reshape(1, 100),
                   Wo, bo.reshape(1, 1))
    return out[:N]


# speedup vs baseline: 24.6678x; 1.6173x over previous
import functools

import jax
import jax.numpy as jnp
from jax import lax
from jax.experimental import pallas as pl
from jax.experimental.pallas import tpu as pltpu
from jax.experimental.pallas import tpu_sc as plsc

N = 10000
E = 320000
D = 128
NP = 10240
ROWS_PER_TILE = NP // 16
C = 128
NW = 32
CPW = 80
NCH = NW * CPW
EPAD = NCH * C - E


NBUF = 2
IHALF = CPW // 2
DEG_K = 8


def _deg_body(ei3_hbm, ones_hbm, zd_hbm, deg_out, didx_all, ones_v, deg_sh,
              sem):
    c = lax.axis_index("c")
    s = lax.axis_index("s")
    c0 = (s * 2 + c) * CPW
    pltpu.sync_copy(ones_hbm, ones_v)
    pltpu.sync_copy(zd_hbm, deg_sh.at[pl.ds(s * ROWS_PER_TILE, ROWS_PER_TILE)])
    pltpu.sync_copy(ei3_hbm.at[1, pl.ds(c0, CPW)], didx_all)
    plsc.subcore_barrier()

    def group(gi, carry):
        for b in range(DEG_K):
            pltpu.async_copy(ones_v, deg_sh.at[didx_all.at[gi * DEG_K + b]],
                             sem, add=True)
        for b in range(DEG_K):
            pltpu.make_async_copy(ones_v, deg_sh.at[didx_all.at[0]],
                                  sem).wait()
        return carry

    lax.fori_loop(0, CPW // DEG_K, group, 0)
    plsc.subcore_barrier()
    sl = pl.ds(s * ROWS_PER_TILE, ROWS_PER_TILE)
    pltpu.sync_copy(deg_sh.at[sl], deg_out.at[c, sl])


def _agg_body(h_hbm, ei3_hbm, z_hbm, acc_out, sidx_all, didx_all, rows,
              acc_sh, gsem, ssem):
    c = lax.axis_index("c")
    s = lax.axis_index("s")
    c0 = (s * 2 + c) * CPW
    pltpu.sync_copy(z_hbm, acc_sh.at[pl.ds(s * ROWS_PER_TILE, ROWS_PER_TILE)])
    plsc.subcore_barrier()

    for half in range(2):
        pltpu.sync_copy(ei3_hbm.at[0, pl.ds(c0 + half * IHALF, IHALF)],
                        sidx_all)
        pltpu.sync_copy(ei3_hbm.at[1, pl.ds(c0 + half * IHALF, IHALF)],
                        didx_all)
        for b in range(NBUF):
            pltpu.async_copy(h_hbm.at[sidx_all.at[b]], rows.at[b], gsem.at[b])

        def group(gi, carry):
            for b in range(NBUF):
                g = gi * NBUF + b
                pltpu.make_async_copy(h_hbm.at[sidx_all.at[g]], rows.at[b],
                                      gsem.at[b]).wait()
                pltpu.async_copy(rows.at[b], acc_sh.at[didx_all.at[g]],
                                 ssem.at[b], add=True)
            for b in range(NBUF):
                g = gi * NBUF + b
                pltpu.make_async_copy(rows.at[b], acc_sh.at[didx_all.at[0]],
                                      ssem.at[b]).wait()

                @pl.when(g + NBUF < IHALF)
                def _():
                    pltpu.async_copy(h_hbm.at[sidx_all.at[g + NBUF]],
                                     rows.at[b], gsem.at[b])
            return carry

        lax.fori_loop(0, IHALF // NBUF, group, 0)
    plsc.subcore_barrier()
    sl = pl.ds(s * ROWS_PER_TILE, ROWS_PER_TILE)
    pltpu.sync_copy(acc_sh.at[sl], acc_out.at[c, sl])


def _sc_deg(ei3, ones_hbm, zd_hbm):
    mesh = plsc.VectorSubcoreMesh(core_axis_name="c", subcore_axis_name="s")
    f = pl.kernel(
        _deg_body,
        out_type=jax.ShapeDtypeStruct((2, NP), jnp.float32),
        mesh=mesh,
        scratch_types=[
            pltpu.VMEM((CPW, C), jnp.int32),
            pltpu.VMEM((C,), jnp.float32),
            pltpu.VMEM_SHARED((NP,), jnp.float32),
            pltpu.SemaphoreType.DMA,
        ],
    )
    return f(ei3, ones_hbm, zd_hbm)


def _sc_agg(h, ei3, z_hbm):
    mesh = plsc.VectorSubcoreMesh(core_axis_name="c", subcore_axis_name="s")
    f = pl.kernel(
        _agg_body,
        out_type=jax.ShapeDtypeStruct((2, NP, D), jnp.float32),
        mesh=mesh,
        scratch_types=[
            pltpu.VMEM((IHALF, C), jnp.int32),
            pltpu.VMEM((IHALF, C), jnp.int32),
            pltpu.VMEM((NBUF, C, D), jnp.float32),
            pltpu.VMEM_SHARED((NP, D), jnp.float32),
            pltpu.SemaphoreType.DMA((NBUF,)),
            pltpu.SemaphoreType.DMA((NBUF,)),
        ],
    )
    return f(h, ei3, z_hbm)


BLK = 1024
GRID = NP // BLK


def _swish(v):
    return v * jax.nn.sigmoid(v)


def _k1_body(x_ref, w_ref, r_ref, o_ref):
    o_ref[...] = jnp.dot(x_ref[...], w_ref[...],
                         preferred_element_type=jnp.float32) * r_ref[...]


def _tc_h1(x, W1, r):
    return pl.pallas_call(
        _k1_body,
        grid=(GRID,),
        in_specs=[
            pl.BlockSpec((BLK, D), lambda i: (i, 0)),
            pl.BlockSpec((D, D), lambda i: (0, 0)),
            pl.BlockSpec((BLK, 1), lambda i: (i, 0)),
        ],
        out_specs=pl.BlockSpec((BLK, D), lambda i: (i, 0)),
        out_shape=jax.ShapeDtypeStruct((NP, D), jnp.float32),
    )(x, W1, r)


def _k3_body(acc_ref, h_ref, r_ref, b_ref, w_ref, o_ref):
    agg = (acc_ref[0] + acc_ref[1] + h_ref[...]) * r_ref[...]
    x2 = _swish(agg + b_ref[...])
    o_ref[...] = jnp.dot(x2, w_ref[...],
                         preferred_element_type=jnp.float32) * r_ref[...]


def _tc_mid(acc, h, r, b1, W2):
    return pl.pallas_call(
        _k3_body,
        grid=(GRID,),
        in_specs=[
            pl.BlockSpec((2, BLK, D), lambda i: (0, i, 0)),
            pl.BlockSpec((BLK, D), lambda i: (i, 0)),
            pl.BlockSpec((BLK, 1), lambda i: (i, 0)),
            pl.BlockSpec((1, D), lambda i: (0, 0)),
            pl.BlockSpec((D, D), lambda i: (0, 0)),
        ],
        out_specs=pl.BlockSpec((BLK, D), lambda i: (i, 0)),
        out_shape=jax.ShapeDtypeStruct((NP, D), jnp.float32),
    )(acc, h, r, b1, W2)


def _k5_body(acc_ref, h_ref, r_ref, b_ref, wd_ref, bd_ref, wo_ref, bo_ref,
             o_ref):
    agg = (acc_ref[0] + acc_ref[1] + h_ref[...]) * r_ref[...]
    x3 = _swish(agg + b_ref[...])
    g = _swish(jnp.dot(x3, wd_ref[...],
                       preferred_element_type=jnp.float32) + bd_ref[...])
    o_ref[...] = jax.nn.sigmoid(
        jnp.dot(g, wo_ref[...], preferred_element_type=jnp.float32)
        + bo_ref[...])


def _tc_head(acc, h, r, b2, Wd, bd, Wo, bo):
    return pl.pallas_call(
        _k5_body,
        grid=(GRID,),
        in_specs=[
            pl.BlockSpec((2, BLK, D), lambda i: (0, i, 0)),
            pl.BlockSpec((BLK, D), lambda i: (i, 0)),
            pl.BlockSpec((BLK, 1), lambda i: (i, 0)),
            pl.BlockSpec((1, D), lambda i: (0, 0)),
            pl.BlockSpec((D, 100), lambda i: (0, 0)),
            pl.BlockSpec((1, 100), lambda i: (0, 0)),
            pl.BlockSpec((100, 1), lambda i: (0, 0)),
            pl.BlockSpec((1, 1), lambda i: (0, 0)),
        ],
        out_specs=pl.BlockSpec((BLK, 1), lambda i: (i, 0)),
        out_shape=jax.ShapeDtypeStruct((NP, 1), jnp.float32),
    )(acc, h, r, b2, Wd, bd, Wo, bo)


def kernel(x, edge_index, W1, b1, W2, b2, Wd, bd, Wo, bo):
    x_pad = jnp.zeros((NP, D), jnp.float32).at[:N].set(x)
    pad_idx = N + (jnp.arange(EPAD, dtype=jnp.int32) % (NP - N))
    ei3 = jnp.concatenate(
        [edge_index, jnp.stack([pad_idx, pad_idx])], axis=1
    ).reshape(2, NCH, C)
    ones_hbm = jnp.ones((C,), jnp.float32)
    zd_hbm = jnp.zeros((ROWS_PER_TILE,), jnp.float32)
    z_hbm = jnp.zeros((ROWS_PER_TILE, D), jnp.float32)

    deg2 = _sc_deg(ei3, ones_hbm, zd_hbm)
    r = lax.rsqrt(deg2[0] + deg2[1] + 1.0)[:, None]

    h1 = _tc_h1(x_pad, W1, r)
    acc1 = _sc_agg(h1, ei3, z_hbm)
    h2 = _tc_mid(acc1, h1, r, b1.reshape(1, D), W2)
    acc2 = _sc_agg(h2, ei3, z_hbm)
    out = _tc_head(acc2, h2, r, b2.reshape(1, D), Wd, bd.reshape(1, 100),
                   Wo, bo.reshape(1, 1))
    return out[:N]
